# Initial kernel scaffold; baseline (speedup 1.0000x reference)
import jax
import jax.numpy as jnp
from jax import lax
from jax.experimental import pallas as pl
from jax.experimental.pallas import tpu as pltpu
from jax.experimental.pallas import tpu_sc as plsc

N = 10000
E = 320000
DF = 128
DH = 32

NC = 2
NS = 16
NW = NC * NS
LANE = 128
R = 79
EPAD = NW * R * LANE
NPAD = 10016
STRIPE = NPAD // NS
DW = 16

f32 = jnp.float32

_mesh = plsc.VectorSubcoreMesh(core_axis_name="c", subcore_axis_name="s")


def _deg_body(dst_hbm, ones_hbm, zeros_hbm, out_hbm, dst_v, ones_v, deg_sh):
    cid = lax.axis_index("c")
    sid = lax.axis_index("s")
    wid = cid * NS + sid
    stripe = pl.ds(sid * STRIPE, STRIPE)
    pltpu.sync_copy(zeros_hbm.at[stripe], deg_sh.at[stripe])
    pltpu.sync_copy(ones_hbm, ones_v)
    pltpu.sync_copy(dst_hbm.at[wid], dst_v)
    plsc.subcore_barrier()

    @pl.loop(0, R)
    def _(j):
        pltpu.sync_copy(ones_v, deg_sh.at[dst_v.at[j]], add=True)

    plsc.subcore_barrier()
    pltpu.sync_copy(deg_sh.at[stripe], out_hbm.at[cid, stripe])


_deg_kernel = pl.kernel(
    _deg_body,
    out_type=jax.ShapeDtypeStruct((NC, NPAD, DW), f32),
    mesh=_mesh,
    scratch_types=[
        pltpu.VMEM((R, LANE), jnp.int32),
        pltpu.VMEM((LANE, DW), f32),
        pltpu.VMEM_SHARED((NPAD, DW), f32),
    ],
)


def _gs_body(g_hbm, src_hbm, dst_hbm, zeros_hbm, out_hbm,
             src_v, dst_v, rows_v, acc_sh):
    cid = lax.axis_index("c")
    sid = lax.axis_index("s")
    wid = cid * NS + sid
    stripe = pl.ds(sid * STRIPE, STRIPE)
    pltpu.sync_copy(zeros_hbm.at[stripe], acc_sh.at[stripe])
    pltpu.sync_copy(src_hbm.at[wid], src_v)
    pltpu.sync_copy(dst_hbm.at[wid], dst_v)
    plsc.subcore_barrier()

    @pl.loop(0, R)
    def _(j):
        pltpu.sync_copy(g_hbm.at[src_v.at[j]], rows_v)
        pltpu.sync_copy(rows_v, acc_sh.at[dst_v.at[j]], add=True)

    plsc.subcore_barrier()
    pltpu.sync_copy(acc_sh.at[stripe], out_hbm.at[cid, stripe])


_gs_kernel = pl.kernel(
    _gs_body,
    out_type=jax.ShapeDtypeStruct((NC, NPAD, DH), f32),
    mesh=_mesh,
    scratch_types=[
        pltpu.VMEM((R, LANE), jnp.int32),
        pltpu.VMEM((R, LANE), jnp.int32),
        pltpu.VMEM((LANE, DH), f32),
        pltpu.VMEM_SHARED((NPAD, DH), f32),
    ],
)


def _mm1_body(x_ref, w1_ref, h_ref):
    h_ref[...] = jnp.dot(x_ref[...], w1_ref[...], preferred_element_type=f32)


_mm1 = pl.pallas_call(
    _mm1_body,
    out_shape=jax.ShapeDtypeStruct((NPAD, DH), f32),
)


def _scale1_body(degp_ref, h_ref, g1_ref, dinv_ref):
    deg = degp_ref[0, :, 0:1] + degp_ref[1, :, 0:1] + 1.0
    dinv = lax.rsqrt(deg)
    g1_ref[...] = h_ref[...] * dinv
    dinv_ref[...] = dinv


_scale1 = pl.pallas_call(
    _scale1_body,
    out_shape=(
        jax.ShapeDtypeStruct((NPAD, DH), f32),
        jax.ShapeDtypeStruct((NPAD, 1), f32),
    ),
)


def _fin1_body(accp_ref, g1_ref, dinv_ref, b1_ref, gm1_ref, bt1_ref, w2_ref,
               g2_ref):
    s = (accp_ref[0, :N, :] + accp_ref[1, :N, :] + g1_ref[:N, :]) \
        * dinv_ref[:N, :] + b1_ref[...]
    mu = jnp.mean(s, axis=0, keepdims=True)
    var = jnp.mean(jnp.square(s - mu), axis=0, keepdims=True)
    y = jnp.maximum((s - mu) * lax.rsqrt(var + 1e-5) * gm1_ref[...]
                    + bt1_ref[...], 0.0)
    h2 = jnp.dot(y, w2_ref[...], preferred_element_type=f32)
    g2_ref[:N, :] = h2 * dinv_ref[:N, :]
    g2_ref[N:, :] = jnp.zeros((NPAD - N, DH), f32)


_fin1 = pl.pallas_call(
    _fin1_body,
    out_shape=jax.ShapeDtypeStruct((NPAD, DH), f32),
)


def _fin2_body(accp_ref, g2_ref, dinv_ref, b2_ref, gm2_ref, bt2_ref, o_ref):
    s = (accp_ref[0, :N, :] + accp_ref[1, :N, :] + g2_ref[:N, :]) \
        * dinv_ref[:N, :] + b2_ref[...]
    mu = jnp.mean(s, axis=0, keepdims=True)
    var = jnp.mean(jnp.square(s - mu), axis=0, keepdims=True)
    o_ref[...] = (s - mu) * lax.rsqrt(var + 1e-5) * gm2_ref[...] + bt2_ref[...]


_fin2 = pl.pallas_call(
    _fin2_body,
    out_shape=jax.ShapeDtypeStruct((N, DH), f32),
)


def kernel(x, edge_index, W1, b1, gamma1, beta1, W2, b2, gamma2, beta2):
    pad_e = EPAD - E
    src = jnp.concatenate(
        [edge_index[0], jnp.zeros((pad_e,), jnp.int32)]).reshape(NW, R, LANE)
    dst = jnp.concatenate(
        [edge_index[1], jnp.full((pad_e,), N, jnp.int32)]).reshape(NW, R, LANE)
    x_p = jnp.pad(x, ((0, NPAD - N), (0, 0)))
    zeros_dw = jnp.zeros((NPAD, DW), f32)
    zeros_dh = jnp.zeros((NPAD, DH), f32)
    ones_dw = jnp.ones((LANE, DW), f32)

    degp = _deg_kernel(dst, ones_dw, zeros_dw)
    h1 = _mm1(x_p, W1)
    g1, dinv = _scale1(degp, h1)
    acc1 = _gs_kernel(g1, src, dst, zeros_dh)
    g2 = _fin1(acc1, g1, dinv, b1.reshape(1, DH), gamma1.reshape(1, DH),
               beta1.reshape(1, DH), W2)
    acc2 = _gs_kernel(g2, src, dst, zeros_dh)
    out = _fin2(acc2, g2, dinv, b2.reshape(1, DH), gamma2.reshape(1, DH),
                beta2.reshape(1, DH))
    return out


# speedup vs baseline: 25.4445x; 25.4445x over previous
import jax
import jax.numpy as jnp
from jax import lax
from jax.experimental import pallas as pl
from jax.experimental.pallas import tpu as pltpu
from jax.experimental.pallas import tpu_sc as plsc

N = 10000
E = 320000
DF = 128
DH = 32

NC = 2
NS = 16
NW = NC * NS
LANE = 128
R = 79
EPAD = NW * R * LANE
NPAD = 10112
STRIPE = NPAD // NS
DW = 16

f32 = jnp.float32

_mesh = plsc.VectorSubcoreMesh(core_axis_name="c", subcore_axis_name="s")
_sc_params = pltpu.CompilerParams(use_tc_tiling_on_sc=False)


def _deg_body(dst_hbm, ones_hbm, zeros_hbm, out_hbm, dst_v, ones_v, deg_sh):
    cid = lax.axis_index("c")
    sid = lax.axis_index("s")
    wid = cid * NS + sid
    stripe = pl.ds(sid * STRIPE, STRIPE)
    pltpu.sync_copy(zeros_hbm.at[stripe], deg_sh.at[stripe])
    pltpu.sync_copy(ones_hbm, ones_v)
    pltpu.sync_copy(dst_hbm.at[wid], dst_v)
    plsc.subcore_barrier()

    @pl.loop(0, R)
    def _(j):
        pltpu.sync_copy(ones_v, deg_sh.at[dst_v.at[j]], add=True)

    plsc.subcore_barrier()
    pltpu.sync_copy(deg_sh.at[stripe], out_hbm.at[cid, stripe])


_deg_kernel = pl.kernel(
    _deg_body,
    out_type=jax.ShapeDtypeStruct((NC, NPAD, DW), f32),
    mesh=_mesh,
    scratch_types=[
        pltpu.VMEM((R, LANE), jnp.int32),
        pltpu.VMEM((LANE, DW), f32),
        pltpu.VMEM_SHARED((NPAD, DW), f32),
    ],
    compiler_params=_sc_params,
)


def _gs_body(g_hbm, src_hbm, dst_hbm, zeros_hbm, out_hbm,
             src_v, dst_v, rows_v, acc_sh):
    cid = lax.axis_index("c")
    sid = lax.axis_index("s")
    wid = cid * NS + sid
    stripe = pl.ds(sid * STRIPE, STRIPE)
    pltpu.sync_copy(zeros_hbm.at[stripe], acc_sh.at[stripe])
    pltpu.sync_copy(src_hbm.at[wid], src_v)
    pltpu.sync_copy(dst_hbm.at[wid], dst_v)
    plsc.subcore_barrier()

    @pl.loop(0, R)
    def _(j):
        pltpu.sync_copy(g_hbm.at[src_v.at[j]], rows_v)
        pltpu.sync_copy(rows_v, acc_sh.at[dst_v.at[j]], add=True)

    plsc.subcore_barrier()
    pltpu.sync_copy(acc_sh.at[stripe], out_hbm.at[cid, stripe])


_gs_kernel = pl.kernel(
    _gs_body,
    out_type=jax.ShapeDtypeStruct((NC, NPAD, DH), f32),
    mesh=_mesh,
    scratch_types=[
        pltpu.VMEM((R, LANE), jnp.int32),
        pltpu.VMEM((R, LANE), jnp.int32),
        pltpu.VMEM((LANE, DH), f32),
        pltpu.VMEM_SHARED((NPAD, DH), f32),
    ],
    compiler_params=_sc_params,
)


def _mm1_body(x_ref, w1_ref, h_ref):
    h_ref[...] = jnp.dot(x_ref[...], w1_ref[...], preferred_element_type=f32)


_mm1 = pl.pallas_call(
    _mm1_body,
    out_shape=jax.ShapeDtypeStruct((NPAD, DH), f32),
)


def _scale1_body(degp_ref, h_ref, g1_ref, dinv_ref):
    deg = degp_ref[0, :, 0:1] + degp_ref[1, :, 0:1] + 1.0
    dinv = lax.rsqrt(deg)
    g1_ref[...] = h_ref[...] * dinv
    dinv_ref[...] = dinv


_scale1 = pl.pallas_call(
    _scale1_body,
    out_shape=(
        jax.ShapeDtypeStruct((NPAD, DH), f32),
        jax.ShapeDtypeStruct((NPAD, 1), f32),
    ),
)


def _fin1_body(accp_ref, g1_ref, dinv_ref, b1_ref, gm1_ref, bt1_ref, w2_ref,
               g2_ref):
    s = (accp_ref[0, :N, :] + accp_ref[1, :N, :] + g1_ref[:N, :]) \
        * dinv_ref[:N, :] + b1_ref[...]
    mu = jnp.mean(s, axis=0, keepdims=True)
    var = jnp.mean(jnp.square(s - mu), axis=0, keepdims=True)
    y = jnp.maximum((s - mu) * lax.rsqrt(var + 1e-5) * gm1_ref[...]
                    + bt1_ref[...], 0.0)
    h2 = jnp.dot(y, w2_ref[...], preferred_element_type=f32)
    g2_ref[:N, :] = h2 * dinv_ref[:N, :]
    g2_ref[N:, :] = jnp.zeros((NPAD - N, DH), f32)


_fin1 = pl.pallas_call(
    _fin1_body,
    out_shape=jax.ShapeDtypeStruct((NPAD, DH), f32),
)


def _fin2_body(accp_ref, g2_ref, dinv_ref, b2_ref, gm2_ref, bt2_ref, o_ref):
    s = (accp_ref[0, :N, :] + accp_ref[1, :N, :] + g2_ref[:N, :]) \
        * dinv_ref[:N, :] + b2_ref[...]
    mu = jnp.mean(s, axis=0, keepdims=True)
    var = jnp.mean(jnp.square(s - mu), axis=0, keepdims=True)
    o_ref[...] = (s - mu) * lax.rsqrt(var + 1e-5) * gm2_ref[...] + bt2_ref[...]


_fin2 = pl.pallas_call(
    _fin2_body,
    out_shape=jax.ShapeDtypeStruct((N, DH), f32),
)


def kernel(x, edge_index, W1, b1, gamma1, beta1, W2, b2, gamma2, beta2):
    pad_e = EPAD - E
    src = jnp.concatenate(
        [edge_index[0], jnp.zeros((pad_e,), jnp.int32)]).reshape(NW, R, LANE)
    dst = jnp.concatenate(
        [edge_index[1], jnp.full((pad_e,), N, jnp.int32)]).reshape(NW, R, LANE)
    x_p = jnp.pad(x, ((0, NPAD - N), (0, 0)))
    zeros_dw = jnp.zeros((NPAD, DW), f32)
    zeros_dh = jnp.zeros((NPAD, DH), f32)
    ones_dw = jnp.ones((LANE, DW), f32)

    degp = _deg_kernel(dst, ones_dw, zeros_dw)
    h1 = _mm1(x_p, W1)
    g1, dinv = _scale1(degp, h1)
    acc1 = _gs_kernel(g1, src, dst, zeros_dh)
    g2 = _fin1(acc1, g1, dinv, b1.reshape(1, DH), gamma1.reshape(1, DH),
               beta1.reshape(1, DH), W2)
    acc2 = _gs_kernel(g2, src, dst, zeros_dh)
    out = _fin2(acc2, g2, dinv, b2.reshape(1, DH), gamma2.reshape(1, DH),
                beta2.reshape(1, DH))
    return out


# speedup vs baseline: 45.3737x; 1.7832x over previous
import jax
import jax.numpy as jnp
from jax import lax
from jax.experimental import pallas as pl
from jax.experimental.pallas import tpu as pltpu
from jax.experimental.pallas import tpu_sc as plsc

N = 10000
E = 320000
DF = 128
DH = 32

NC = 2
NS = 16
NW = NC * NS
LANE = 512
R = 20
EPAD = NW * R * LANE
NPAD = 10112
STRIPE = NPAD // NS
DW = 16

f32 = jnp.float32

_mesh = plsc.VectorSubcoreMesh(core_axis_name="c", subcore_axis_name="s")
_sc_params = pltpu.CompilerParams(use_tc_tiling_on_sc=False)


def _deg_body(dst_hbm, ones_hbm, zeros_hbm, out_hbm, dst_v, ones_v, deg_sh):
    cid = lax.axis_index("c")
    sid = lax.axis_index("s")
    wid = cid * NS + sid
    stripe = pl.ds(sid * STRIPE, STRIPE)
    pltpu.sync_copy(zeros_hbm.at[stripe], deg_sh.at[stripe])
    pltpu.sync_copy(ones_hbm, ones_v)
    pltpu.sync_copy(dst_hbm.at[wid], dst_v)
    plsc.subcore_barrier()

    for j in range(R):
        pltpu.sync_copy(ones_v, deg_sh.at[dst_v.at[j]], add=True)

    plsc.subcore_barrier()
    pltpu.sync_copy(deg_sh.at[stripe], out_hbm.at[cid, stripe])


_deg_kernel = pl.kernel(
    _deg_body,
    out_type=jax.ShapeDtypeStruct((NC, NPAD, DW), f32),
    mesh=_mesh,
    scratch_types=[
        pltpu.VMEM((R, LANE), jnp.int32),
        pltpu.VMEM((LANE, DW), f32),
        pltpu.VMEM_SHARED((NPAD, DW), f32),
    ],
    compiler_params=_sc_params,
)


def _gs_body(g_hbm, src_hbm, dst_hbm, zeros_hbm, out_hbm,
             src_v, dst_v, rows_a, rows_b, g_sh, acc_sh, sem_g):
    cid = lax.axis_index("c")
    sid = lax.axis_index("s")
    wid = cid * NS + sid
    stripe = pl.ds(sid * STRIPE, STRIPE)
    pltpu.sync_copy(zeros_hbm.at[stripe], acc_sh.at[stripe])
    pltpu.sync_copy(g_hbm.at[stripe], g_sh.at[stripe])
    pltpu.sync_copy(src_hbm.at[wid], src_v)
    pltpu.sync_copy(dst_hbm.at[wid], dst_v)
    plsc.subcore_barrier()

    bufs = (rows_a, rows_b)
    pltpu.async_copy(g_sh.at[src_v.at[0]], rows_a, sem_g)
    for j in range(R):
        buf = bufs[j % 2]
        pltpu.make_async_copy(g_sh.at[src_v.at[j]], buf, sem_g).wait()
        if j + 1 < R:
            pltpu.async_copy(g_sh.at[src_v.at[j + 1]], bufs[(j + 1) % 2],
                             sem_g)
        pltpu.sync_copy(buf, acc_sh.at[dst_v.at[j]], add=True)

    plsc.subcore_barrier()
    pltpu.sync_copy(acc_sh.at[stripe], out_hbm.at[cid, stripe])


_gs_kernel = pl.kernel(
    _gs_body,
    out_type=jax.ShapeDtypeStruct((NC, NPAD, DH), f32),
    mesh=_mesh,
    scratch_types=[
        pltpu.VMEM((R, LANE), jnp.int32),
        pltpu.VMEM((R, LANE), jnp.int32),
        pltpu.VMEM((LANE, DH), f32),
        pltpu.VMEM((LANE, DH), f32),
        pltpu.VMEM_SHARED((NPAD, DH), f32),
        pltpu.VMEM_SHARED((NPAD, DH), f32),
        pltpu.SemaphoreType.DMA,
    ],
    compiler_params=_sc_params,
)


def _mm1_body(x_ref, w1_ref, h_ref):
    h_ref[...] = jnp.dot(x_ref[...], w1_ref[...], preferred_element_type=f32)


_mm1 = pl.pallas_call(
    _mm1_body,
    out_shape=jax.ShapeDtypeStruct((NPAD, DH), f32),
)


def _scale1_body(degp_ref, h_ref, g1_ref, dinv_ref):
    deg = degp_ref[0, :, 0:1] + degp_ref[1, :, 0:1] + 1.0
    dinv = lax.rsqrt(deg)
    g1_ref[...] = h_ref[...] * dinv
    dinv_ref[...] = dinv


_scale1 = pl.pallas_call(
    _scale1_body,
    out_shape=(
        jax.ShapeDtypeStruct((NPAD, DH), f32),
        jax.ShapeDtypeStruct((NPAD, 1), f32),
    ),
)


def _fin1_body(accp_ref, g1_ref, dinv_ref, b1_ref, gm1_ref, bt1_ref, w2_ref,
               g2_ref):
    s = (accp_ref[0, :N, :] + accp_ref[1, :N, :] + g1_ref[:N, :]) \
        * dinv_ref[:N, :] + b1_ref[...]
    mu = jnp.mean(s, axis=0, keepdims=True)
    var = jnp.mean(jnp.square(s - mu), axis=0, keepdims=True)
    y = jnp.maximum((s - mu) * lax.rsqrt(var + 1e-5) * gm1_ref[...]
                    + bt1_ref[...], 0.0)
    h2 = jnp.dot(y, w2_ref[...], preferred_element_type=f32)
    g2_ref[:N, :] = h2 * dinv_ref[:N, :]
    g2_ref[N:, :] = jnp.zeros((NPAD - N, DH), f32)


_fin1 = pl.pallas_call(
    _fin1_body,
    out_shape=jax.ShapeDtypeStruct((NPAD, DH), f32),
)


def _fin2_body(accp_ref, g2_ref, dinv_ref, b2_ref, gm2_ref, bt2_ref, o_ref):
    s = (accp_ref[0, :N, :] + accp_ref[1, :N, :] + g2_ref[:N, :]) \
        * dinv_ref[:N, :] + b2_ref[...]
    mu = jnp.mean(s, axis=0, keepdims=True)
    var = jnp.mean(jnp.square(s - mu), axis=0, keepdims=True)
    o_ref[...] = (s - mu) * lax.rsqrt(var + 1e-5) * gm2_ref[...] + bt2_ref[...]


_fin2 = pl.pallas_call(
    _fin2_body,
    out_shape=jax.ShapeDtypeStruct((N, DH), f32),
)


def kernel(x, edge_index, W1, b1, gamma1, beta1, W2, b2, gamma2, beta2):
    pad_e = EPAD - E
    src = jnp.concatenate(
        [edge_index[0], jnp.zeros((pad_e,), jnp.int32)]).reshape(NW, R, LANE)
    dst = jnp.concatenate(
        [edge_index[1], jnp.full((pad_e,), N, jnp.int32)]).reshape(NW, R, LANE)
    x_p = jnp.pad(x, ((0, NPAD - N), (0, 0)))
    zeros_dw = jnp.zeros((NPAD, DW), f32)
    zeros_dh = jnp.zeros((NPAD, DH), f32)
    ones_dw = jnp.ones((LANE, DW), f32)

    degp = _deg_kernel(dst, ones_dw, zeros_dw)
    h1 = _mm1(x_p, W1)
    g1, dinv = _scale1(degp, h1)
    acc1 = _gs_kernel(g1, src, dst, zeros_dh)
    g2 = _fin1(acc1, g1, dinv, b1.reshape(1, DH), gamma1.reshape(1, DH),
               beta1.reshape(1, DH), W2)
    acc2 = _gs_kernel(g2, src, dst, zeros_dh)
    out = _fin2(acc2, g2, dinv, b2.reshape(1, DH), gamma2.reshape(1, DH),
                beta2.reshape(1, DH))
    return out
